# baseline (speedup 1.0000x reference)
import functools

import jax
import jax.numpy as jnp
from jax import lax
from jax.experimental import pallas as pl
from jax.experimental.pallas import tpu as pltpu
from jax.experimental.pallas import tpu_sc as plsc

N = 10000
NP = 10240
E = 320000
EC = 64000
K = 128
NT = 16
TROWS = NP // NT
WB = TROWS // K
PT = 20480
PTC16 = 4096
PTC32 = 2048

f32 = jnp.float32


def _mesh():
    return plsc.VectorSubcoreMesh(core_axis_name="c", subcore_axis_name="s")


def _zero_fill(buf, rows, cols):
    z = jnp.zeros((16,), f32)

    def body(r, carry):
        for j in range(cols // 16):
            buf[r, pl.ds(j * 16, 16)] = z
        return carry

    lax.fori_loop(0, rows, body, 0)


def _zero_fill_1d(buf, n):
    z = jnp.zeros((16,), f32)

    def body(r, carry):
        buf[pl.ds(r * 16, 16)] = z
        return carry

    lax.fori_loop(0, n // 16, body, 0)


def _spmem_zero(acc, zbuf, sid):
    off = sid * TROWS
    for b in range(WB):
        pltpu.sync_copy(zbuf, acc.at[pl.ds(off + b * K, K)])


NB = 2
SUP = 16
PIPE = True


def _edge_pass(row2d, col2d, ew2d, table, acc, rbuf, cbuf, ebuf, slots,
               gsem, isem, tile, nchunks, d):
    cb = tile * nchunks
    nsup = nchunks // SUP

    if not PIPE:

        def do_super_sync(s, carry):
            sb = lax.rem(s, 2)
            base = cb + s * SUP
            pltpu.sync_copy(row2d.at[pl.ds(base, SUP)], rbuf.at[sb])
            pltpu.sync_copy(col2d.at[pl.ds(base, SUP)], cbuf.at[sb])
            if ew2d is not None:
                pltpu.sync_copy(ew2d.at[pl.ds(base, SUP)], ebuf.at[sb])

            def chunk(j, c2):
                pltpu.sync_copy(table.at[rbuf.at[sb, j]], slots.at[0])
                if ew2d is not None:

                    def scale(g, c3):
                        vew = ebuf[sb, j, pl.ds(g * 16, 16)]
                        for r in range(16):
                            sc = vew[r]
                            row = g * 16 + r
                            for q in range(d // 16):
                                slots[0, row, pl.ds(q * 16, 16)] = (
                                    slots[0, row, pl.ds(q * 16, 16)] * sc)
                        return c3

                    lax.fori_loop(0, K // 16, scale, 0)
                pltpu.sync_copy(slots.at[0], acc.at[cbuf.at[sb, j]],
                                add=True)
                return c2

            lax.fori_loop(0, SUP, chunk, 0)
            return carry

        lax.fori_loop(0, nsup, do_super_sync, 0)
        return

    def idx_issue(s, sb):
        base = cb + s * SUP
        pltpu.async_copy(row2d.at[pl.ds(base, SUP)], rbuf.at[sb], isem)
        pltpu.async_copy(col2d.at[pl.ds(base, SUP)], cbuf.at[sb], isem)
        if ew2d is not None:
            pltpu.async_copy(ew2d.at[pl.ds(base, SUP)], ebuf.at[sb], isem)

    def idx_wait(sb):
        pltpu.make_async_copy(row2d.at[pl.ds(cb, SUP)], rbuf.at[sb],
                              isem).wait()
        pltpu.make_async_copy(col2d.at[pl.ds(cb, SUP)], cbuf.at[sb],
                              isem).wait()
        if ew2d is not None:
            pltpu.make_async_copy(ew2d.at[pl.ds(cb, SUP)], ebuf.at[sb],
                                  isem).wait()

    for s in range(min(2, nsup)):
        idx_issue(s, s)

    def do_super(s, carry):
        sb = lax.rem(s, 2)
        idx_wait(sb)
        for b in range(NB):
            pltpu.async_copy(table.at[rbuf.at[sb, b]], slots.at[b], gsem)

        def chunk(j, c2):
            slot = lax.rem(j, NB)
            pltpu.make_async_copy(table.at[rbuf.at[sb, j]],
                                  slots.at[slot], gsem).wait()
            if ew2d is not None:

                def scale(g, c3):
                    vew = ebuf[sb, j, pl.ds(g * 16, 16)]
                    for r in range(16):
                        sc = vew[r]
                        row = g * 16 + r
                        for q in range(d // 16):
                            slots[slot, row, pl.ds(q * 16, 16)] = (
                                slots[slot, row, pl.ds(q * 16, 16)] * sc)
                    return c3

                lax.fori_loop(0, K // 16, scale, 0)
            pltpu.sync_copy(slots.at[slot], acc.at[cbuf.at[sb, j]],
                            add=True)

            @pl.when(j + NB < SUP)
            def _():
                pltpu.async_copy(table.at[rbuf.at[sb, j + NB]],
                                 slots.at[slot], gsem)

            return c2

        lax.fori_loop(0, SUP, chunk, 0)

        @pl.when(s + 2 < nsup)
        def _():
            idx_issue(s + 2, sb)

        return carry

    lax.fori_loop(0, nsup, do_super, 0)


def _writeback(acc, out_hbm, bounce, sid):
    off = sid * TROWS
    for b in range(WB):
        pltpu.sync_copy(acc.at[pl.ds(off + b * K, K)], bounce)
        pltpu.sync_copy(bounce, out_hbm.at[pl.ds(off + b * K, K)])


def _deg_body(sim_col, sim_ew, dist_col, dist_ew, com_col, com_ew,
              deg_s, deg_d, deg_c, dega, degb, cidx, vals, bounce, ssem):
    cid = lax.axis_index("c")
    sid = lax.axis_index("s")

    _zero_fill_1d(bounce, TROWS)
    pltpu.sync_copy(bounce, dega.at[pl.ds(sid * TROWS, TROWS)])
    pltpu.sync_copy(bounce, degb.at[pl.ds(sid * TROWS, TROWS)])
    plsc.subcore_barrier()

    DK = 8

    def scalar_pass(col2d, ew2d, acc, nchunks):
        pltpu.sync_copy(col2d.at[pl.ds(sid * nchunks, nchunks)],
                        cidx.at[pl.ds(0, nchunks)])
        pltpu.sync_copy(ew2d.at[pl.ds(sid * nchunks, nchunks)],
                        vals.at[pl.ds(0, nchunks)])

        def chunk(i, carry):
            pltpu.async_copy(vals.at[i], acc.at[cidx.at[i]], ssem,
                             add=True)

            @pl.when(i >= DK)
            def _():
                pltpu.make_async_copy(vals.at[0], acc.at[cidx.at[0]],
                                      ssem).wait()

            return carry

        lax.fori_loop(0, nchunks, chunk, 0)

        def drain(i, carry):
            pltpu.make_async_copy(vals.at[0], acc.at[cidx.at[0]],
                                  ssem).wait()
            return carry

        lax.fori_loop(0, DK, drain, 0)

    @pl.when(cid == 0)
    def _():
        scalar_pass(sim_col, sim_ew, dega, PT // K)

    @pl.when(cid == 1)
    def _():
        scalar_pass(dist_col, dist_ew, dega, PT // K)
        scalar_pass(com_col, com_ew, degb, PTC16 // K)

    plsc.subcore_barrier()
    off = sid * TROWS

    @pl.when(cid == 0)
    def _():
        pltpu.sync_copy(dega.at[pl.ds(off, TROWS)], bounce)
        pltpu.sync_copy(bounce, deg_s.at[pl.ds(off, TROWS)])

    @pl.when(cid == 1)
    def _():
        pltpu.sync_copy(dega.at[pl.ds(off, TROWS)], bounce)
        pltpu.sync_copy(bounce, deg_d.at[pl.ds(off, TROWS)])
        pltpu.sync_copy(degb.at[pl.ds(off, TROWS)], bounce)
        pltpu.sync_copy(bounce, deg_c.at[pl.ds(off, TROWS)])


def _sc_deg(sim_col, sim_ew, dist_col, dist_ew, com_col, com_ew):
    return pl.kernel(
        _deg_body,
        out_type=(jax.ShapeDtypeStruct((NP,), f32),
                  jax.ShapeDtypeStruct((NP,), f32),
                  jax.ShapeDtypeStruct((NP,), f32)),
        mesh=_mesh(),
        scratch_types=[
            pltpu.VMEM_SHARED((NP,), f32),
            pltpu.VMEM_SHARED((NP,), f32),
            pltpu.VMEM((PT // K, K), jnp.int32),
            pltpu.VMEM((PT // K, K), f32),
            pltpu.VMEM((TROWS,), f32),
            pltpu.SemaphoreType.DMA,
        ],
    )(sim_col, sim_ew, dist_col, dist_ew, com_col, com_ew)


def _l1_body(sim_row, sim_col, sim_ew, dist_row, dist_col, y1, y2,
             agg_sim, agg_dist, acc, rbuf, cbuf, ebuf, slots, gsem, isem):
    cid = lax.axis_index("c")
    sid = lax.axis_index("s")

    zb = slots.at[0]
    _zero_fill(zb, K, 128)
    _spmem_zero(acc, zb, sid)
    plsc.subcore_barrier()

    nchunks = PT // K

    @pl.when(cid == 0)
    def _():
        _edge_pass(sim_row, sim_col, sim_ew, y1, acc, rbuf, cbuf, ebuf,
                   slots, gsem, isem, sid, nchunks, 128)

    @pl.when(cid == 1)
    def _():
        _edge_pass(dist_row, dist_col, None, y2, acc, rbuf, cbuf, ebuf,
                   slots, gsem, isem, sid, nchunks, 128)

    plsc.subcore_barrier()

    @pl.when(cid == 0)
    def _():
        _writeback(acc, agg_sim, zb, sid)

    @pl.when(cid == 1)
    def _():
        _writeback(acc, agg_dist, zb, sid)


def _sc_l1(sim_row, sim_col, sim_ew, dist_row, dist_col, y1, y2):
    return pl.kernel(
        _l1_body,
        out_type=(jax.ShapeDtypeStruct((NP, 128), f32),
                  jax.ShapeDtypeStruct((NP, 128), f32)),
        mesh=_mesh(),
        scratch_types=[
            pltpu.VMEM_SHARED((NP, 128), f32),
            pltpu.VMEM((2, SUP, K), jnp.int32),
            pltpu.VMEM((2, SUP, K), jnp.int32),
            pltpu.VMEM((2, SUP, K), f32),
            pltpu.VMEM((NB, K, 128), f32),
            pltpu.SemaphoreType.DMA,
            pltpu.SemaphoreType.DMA,
        ],
    )(sim_row, sim_col, sim_ew, dist_row, dist_col, y1, y2)


def _l2_body(sim_row, sim_col, sim_ew, dist_row, dist_col, com_row,
             com_col, y3, y4, u, agg3, agg4, aggc0, aggc1,
             acc32, acc64, rbuf, cbuf, ebuf, slots32, slots64, gsem,
             isem):
    cid = lax.axis_index("c")
    sid = lax.axis_index("s")

    zb32 = slots32.at[0]
    zb64 = slots64.at[0]
    _zero_fill(zb32, K, 32)
    _zero_fill(zb64, K, 64)
    _spmem_zero(acc32, zb32, sid)
    _spmem_zero(acc64, zb64, sid)
    plsc.subcore_barrier()

    nchunks = PT // K

    @pl.when(cid == 0)
    def _():
        _edge_pass(sim_row, sim_col, sim_ew, y3, acc32, rbuf, cbuf,
                   ebuf, slots32, gsem, isem, sid, nchunks, 32)

    @pl.when(cid == 1)
    def _():
        _edge_pass(dist_row, dist_col, None, y4, acc32, rbuf, cbuf,
                   ebuf, slots32, gsem, isem, sid, nchunks, 32)

    ctile = cid * NT + sid
    _edge_pass(com_row, com_col, None, u, acc64, rbuf, cbuf, None,
               slots64, gsem, isem, ctile, PTC32 // K, 64)

    plsc.subcore_barrier()

    @pl.when(cid == 0)
    def _():
        _writeback(acc32, agg3, zb32, sid)
        _writeback(acc64, aggc0, zb64, sid)

    @pl.when(cid == 1)
    def _():
        _writeback(acc32, agg4, zb32, sid)
        _writeback(acc64, aggc1, zb64, sid)


def _sc_l2(sim_row, sim_col, sim_ew, dist_row, dist_col, com_row,
           com_col, y3, y4, u):
    return pl.kernel(
        _l2_body,
        out_type=(jax.ShapeDtypeStruct((NP, 32), f32),
                  jax.ShapeDtypeStruct((NP, 32), f32),
                  jax.ShapeDtypeStruct((NP, 64), f32),
                  jax.ShapeDtypeStruct((NP, 64), f32)),
        mesh=_mesh(),
        scratch_types=[
            pltpu.VMEM_SHARED((NP, 32), f32),
            pltpu.VMEM_SHARED((NP, 64), f32),
            pltpu.VMEM((2, SUP, K), jnp.int32),
            pltpu.VMEM((2, SUP, K), jnp.int32),
            pltpu.VMEM((2, SUP, K), f32),
            pltpu.VMEM((NB, K, 32), f32),
            pltpu.VMEM((NB, K, 64), f32),
            pltpu.SemaphoreType.DMA,
            pltpu.SemaphoreType.DMA,
        ],
        compiler_params=pltpu.CompilerParams(use_tc_tiling_on_sc=False),
    )(sim_row, sim_col, sim_ew, dist_row, dist_col, com_row, com_col,
      y3, y4, u)


_MB = NP // 8


def _tc1_body(deg_s, deg_d, deg_c, x, w1, w2, b1, b2, xadt, wpro, xatac,
              watac, bpa, dinv_s, dinv_d, dinv_c, y1, y2, sl1, sl2, u,
              slu):
    ds = lax.rsqrt(jnp.maximum(deg_s[...] + 1.0, 1e-12))
    dd = lax.rsqrt(jnp.maximum(deg_d[...] + 1.0, 1e-12))
    dc = lax.rsqrt(jnp.maximum(deg_c[...] + 1.0, 1e-12))
    dinv_s[...] = ds
    dinv_d[...] = dd
    dinv_c[...] = dc
    xw1 = jnp.dot(x[...], w1[...], preferred_element_type=f32)
    xw2 = jnp.dot(x[...], w2[...], preferred_element_type=f32)
    y1[...] = xw1 * ds
    y2[...] = xw2 * dd
    sl1[...] = xw1 * (ds * ds) + b1[...][None, :]
    sl2[...] = xw2 * (dd * dd) + b2[...][None, :]
    za = jnp.dot(xadt[...], wpro[...], preferred_element_type=f32)
    zt = jnp.dot(xatac[...], watac[...], preferred_element_type=f32)
    zc = jnp.concatenate([za, zt], axis=1)
    u[...] = zc * dc
    slu[...] = zc * (dc * dc) + bpa[...][None, :]


def _tc1(deg_s, deg_d, deg_c, x, w1, w2, b1, b2, xadt, wpro, xatac,
         watac, bpa):
    blk = lambda *s: pl.BlockSpec(s, lambda i: (0,) * len(s))
    row = lambda d: pl.BlockSpec((_MB, d), lambda i: (i, 0))
    return pl.pallas_call(
        _tc1_body,
        grid=(8,),
        in_specs=[
            row(1), row(1), row(1),
            row(128), blk(128, 128), blk(128, 128), blk(128,), blk(128,),
            row(16), blk(16, 32), row(64), blk(64, 32), blk(64,),
        ],
        out_specs=[
            row(1), row(1), row(1),
            row(128), row(128), row(128), row(128), row(64), row(64),
        ],
        out_shape=[
            jax.ShapeDtypeStruct((NP, 1), f32),
            jax.ShapeDtypeStruct((NP, 1), f32),
            jax.ShapeDtypeStruct((NP, 1), f32),
            jax.ShapeDtypeStruct((NP, 128), f32),
            jax.ShapeDtypeStruct((NP, 128), f32),
            jax.ShapeDtypeStruct((NP, 128), f32),
            jax.ShapeDtypeStruct((NP, 128), f32),
            jax.ShapeDtypeStruct((NP, 64), f32),
            jax.ShapeDtypeStruct((NP, 64), f32),
        ],
    )(deg_s, deg_d, deg_c, x, w1, w2, b1, b2, xadt, wpro, xatac, watac,
      bpa)


def _tc2_body(dinv_s, dinv_d, agg_s, agg_d, sl1, sl2, wsim, wdist, bsim,
              bdist, y3, y4, sl3, sl4):
    ds = dinv_s[...]
    dd = dinv_d[...]
    xs = jnp.maximum(agg_s[...] * ds + sl1[...], 0.0)
    xd = jnp.maximum(agg_d[...] * dd + sl2[...], 0.0)
    zs = jnp.dot(xs, wsim[...], preferred_element_type=f32)
    zd = jnp.dot(xd, wdist[...], preferred_element_type=f32)
    y3[...] = zs * ds
    y4[...] = zd * dd
    sl3[...] = zs * (ds * ds) + bsim[...][None, :]
    sl4[...] = zd * (dd * dd) + bdist[...][None, :]


def _tc2(dinv_s, dinv_d, agg_s, agg_d, sl1, sl2, wsim, wdist, bsim,
         bdist):
    blk = lambda *s: pl.BlockSpec(s, lambda i: (0,) * len(s))
    row = lambda d: pl.BlockSpec((_MB, d), lambda i: (i, 0))
    return pl.pallas_call(
        _tc2_body,
        grid=(8,),
        in_specs=[
            row(1), row(1),
            row(128), row(128), row(128), row(128),
            blk(128, 32), blk(128, 32), blk(32,), blk(32,),
        ],
        out_specs=[row(32), row(32), row(32), row(32)],
        out_shape=[jax.ShapeDtypeStruct((NP, 32), f32)] * 4,
    )(dinv_s, dinv_d, agg_s, agg_d, sl1, sl2, wsim, wdist, bsim, bdist)


def _tc3_body(dinv_s, dinv_d, dinv_c, agg3, agg4, aggc0, aggc1, sl3, sl4,
              slu, wf, bf, x_sim, x_dist, fused, pro, atac):
    ds = dinv_s[...]
    dd = dinv_d[...]
    dc = dinv_c[...]
    xsim = agg3[...] * ds + sl3[...]
    xdist = agg4[...] * dd + sl4[...]
    pa = (aggc0[...] + aggc1[...]) * dc + slu[...]
    x_sim[...] = xsim
    x_dist[...] = xdist
    pro[...] = pa[:, :32]
    atac[...] = pa[:, 32:]
    comb = jnp.concatenate([xsim, xdist, pa], axis=1)
    fused[...] = jnp.dot(comb, wf[...], preferred_element_type=f32) \
        + bf[...][None, :]


def _tc3(dinv_s, dinv_d, dinv_c, agg3, agg4, aggc0, aggc1, sl3, sl4, slu,
         wf, bf):
    blk = lambda *s: pl.BlockSpec(s, lambda i: (0,) * len(s))
    row = lambda d: pl.BlockSpec((_MB, d), lambda i: (i, 0))
    return pl.pallas_call(
        _tc3_body,
        grid=(8,),
        in_specs=[
            row(1), row(1), row(1),
            row(32), row(32), row(64), row(64), row(32), row(32), row(64),
            blk(128, 32), blk(32,),
        ],
        out_specs=[row(32), row(32), row(32), row(32), row(32)],
        out_shape=[jax.ShapeDtypeStruct((NP, 32), f32)] * 5,
    )(dinv_s, dinv_d, dinv_c, agg3, agg4, aggc0, aggc1, sl3, sl4, slu,
      wf, bf)


def kernel(x_RNA, x_ADT, x_ATAC, sim_edge_index, sim_edge_weight,
           dist_edge_index, dist_edge_weight, common_edge_index,
           common_edge_weight, W_rna1, b_rna1, W_rna2, b_rna2, W_pro,
           b_pro, W_atac, b_atac, W_sim, b_sim, W_dist, b_dist, W_fuse,
           b_fuse):
    pad = NP - N
    xp = jnp.pad(x_RNA, ((0, pad), (0, 0)))
    xadt = jnp.pad(x_ADT, ((0, pad), (0, 0)))
    xatac = jnp.pad(x_ATAC, ((0, pad), (0, 0)))
    def _tiles(a, ntiles, per, fill):
        orig = a.shape[0] // ntiles
        a2 = jnp.pad(a.reshape(ntiles, orig), ((0, 0), (0, per - orig)),
                     constant_values=fill)
        return a2.reshape(ntiles * (per // K), K)

    sim_row = _tiles(sim_edge_index[0], NT, PT, 0)
    sim_col = _tiles(sim_edge_index[1], NT, PT, NP - 1)
    sim_ew = _tiles(sim_edge_weight, NT, PT, 0.0)
    dist_row = _tiles(dist_edge_index[0], NT, PT, 0)
    dist_col = _tiles(dist_edge_index[1], NT, PT, NP - 1)
    dist_ew = _tiles(dist_edge_weight, NT, PT, 0.0)
    com_row = _tiles(common_edge_index[0], 2 * NT, PTC32, 0)
    com_col = _tiles(common_edge_index[1], 2 * NT, PTC32, NP - 1)
    com_col16 = _tiles(common_edge_index[1], NT, PTC16, NP - 1)
    com_ew16 = _tiles(common_edge_weight, NT, PTC16, 0.0)
    bpa = jnp.concatenate([b_pro, b_atac])

    DBG_DEG = False
    if DBG_DEG:
        deg_s = jnp.zeros((NP,), f32).at[sim_col.reshape(-1)].add(
            sim_ew.reshape(-1))
        deg_d = jnp.zeros((NP,), f32).at[dist_col.reshape(-1)].add(
            dist_ew.reshape(-1))
        deg_c = jnp.zeros((NP,), f32).at[com_col16.reshape(-1)].add(
            com_ew16.reshape(-1))
    else:
        deg_s, deg_d, deg_c = _sc_deg(sim_col, sim_ew, dist_col, dist_ew,
                                      com_col16, com_ew16)

    dinv_s, dinv_d, dinv_c, y1, y2, sl1, sl2, u, slu = _tc1(
        deg_s[:, None], deg_d[:, None], deg_c[:, None], xp, W_rna1,
        W_rna2, b_rna1, b_rna2, xadt, W_pro, xatac, W_atac, bpa)

    agg_sim, agg_dist = _sc_l1(sim_row, sim_col, sim_ew, dist_row,
                               dist_col, y1, y2)

    y3, y4, sl3, sl4 = _tc2(dinv_s, dinv_d, agg_sim, agg_dist, sl1, sl2,
                            W_sim, W_dist, b_sim, b_dist)

    DBG_L2 = False
    if DBG_L2:
        agg3 = jnp.zeros((NP, 32), f32).at[sim_col.reshape(-1)].add(
            y3[sim_row.reshape(-1)] * sim_ew.reshape(-1)[:, None])
        agg4 = jnp.zeros((NP, 32), f32).at[dist_col.reshape(-1)].add(
            y4[dist_row.reshape(-1)])
        aggc0 = jnp.zeros((NP, 64), f32).at[com_col.reshape(-1)].add(
            u[com_row.reshape(-1)])
        aggc1 = jnp.zeros((NP, 64), f32)
    else:
        agg3, agg4, aggc0, aggc1 = _sc_l2(sim_row, sim_col, sim_ew,
                                          dist_row, dist_col, com_row,
                                          com_col, y3, y4, u)

    x_sim, x_dist, fused, pro, atac = _tc3(dinv_s, dinv_d, dinv_c, agg3,
                                           agg4, aggc0, aggc1, sl3, sl4,
                                           slu, W_fuse, b_fuse)

    return (x_sim[:N], x_dist[:N], fused[:N], pro[:N], atac[:N])


# speedup vs baseline: 1.1708x; 1.1708x over previous
import functools

import jax
import jax.numpy as jnp
from jax import lax
from jax.experimental import pallas as pl
from jax.experimental.pallas import tpu as pltpu
from jax.experimental.pallas import tpu_sc as plsc

N = 10000
NP = 10240
E = 320000
EC = 64000
K = 128
NT = 16
TROWS = NP // NT
WB = TROWS // K
PT = 20480
PTC16 = 4096
PTC32 = 2048

f32 = jnp.float32


def _mesh():
    return plsc.VectorSubcoreMesh(core_axis_name="c", subcore_axis_name="s")


def _zero_fill(buf, rows, cols):
    z = jnp.zeros((16,), f32)

    def body(r, carry):
        for j in range(cols // 16):
            buf[r, pl.ds(j * 16, 16)] = z
        return carry

    lax.fori_loop(0, rows, body, 0)


def _zero_fill_1d(buf, n):
    z = jnp.zeros((16,), f32)

    def body(r, carry):
        buf[pl.ds(r * 16, 16)] = z
        return carry

    lax.fori_loop(0, n // 16, body, 0)


def _spmem_zero(acc, zbuf, sid):
    off = sid * TROWS
    for b in range(WB):
        pltpu.sync_copy(zbuf, acc.at[pl.ds(off + b * K, K)])


NB = 2
SUP = 16
PIPE = False


def _edge_pass(row2d, col2d, ew2d, table, acc, rbuf, cbuf, ebuf, slots,
               gsem, isem, tile, nchunks, d):
    cb = tile * nchunks
    nsup = nchunks // SUP

    if not PIPE:

        def do_super_sync(s, carry):
            sb = lax.rem(s, 2)
            base = cb + s * SUP
            pltpu.sync_copy(row2d.at[pl.ds(base, SUP)], rbuf.at[sb])
            pltpu.sync_copy(col2d.at[pl.ds(base, SUP)], cbuf.at[sb])
            if ew2d is not None:
                pltpu.sync_copy(ew2d.at[pl.ds(base, SUP)], ebuf.at[sb])

            def chunk(j, c2):
                pltpu.sync_copy(table.at[rbuf.at[sb, j]], slots.at[0])
                if ew2d is not None:

                    def scale(g, c3):
                        vew = ebuf[sb, j, pl.ds(g * 16, 16)]
                        for r in range(16):
                            sc = vew[r]
                            row = g * 16 + r
                            for q in range(d // 16):
                                slots[0, row, pl.ds(q * 16, 16)] = (
                                    slots[0, row, pl.ds(q * 16, 16)] * sc)
                        return c3

                    lax.fori_loop(0, K // 16, scale, 0)
                pltpu.sync_copy(slots.at[0], acc.at[cbuf.at[sb, j]],
                                add=True)
                return c2

            lax.fori_loop(0, SUP, chunk, 0)
            return carry

        lax.fori_loop(0, nsup, do_super_sync, 0)
        return

    def idx_issue(s, sb):
        base = cb + s * SUP
        pltpu.async_copy(row2d.at[pl.ds(base, SUP)], rbuf.at[sb], isem)
        pltpu.async_copy(col2d.at[pl.ds(base, SUP)], cbuf.at[sb], isem)
        if ew2d is not None:
            pltpu.async_copy(ew2d.at[pl.ds(base, SUP)], ebuf.at[sb], isem)

    def idx_wait(sb):
        pltpu.make_async_copy(row2d.at[pl.ds(cb, SUP)], rbuf.at[sb],
                              isem).wait()
        pltpu.make_async_copy(col2d.at[pl.ds(cb, SUP)], cbuf.at[sb],
                              isem).wait()
        if ew2d is not None:
            pltpu.make_async_copy(ew2d.at[pl.ds(cb, SUP)], ebuf.at[sb],
                                  isem).wait()

    for s in range(min(2, nsup)):
        idx_issue(s, s)

    def do_super(s, carry):
        sb = lax.rem(s, 2)
        idx_wait(sb)
        for b in range(NB):
            pltpu.async_copy(table.at[rbuf.at[sb, b]], slots.at[b], gsem)

        def chunk(j, c2):
            slot = lax.rem(j, NB)
            pltpu.make_async_copy(table.at[rbuf.at[sb, j]],
                                  slots.at[slot], gsem).wait()
            if ew2d is not None:

                def scale(g, c3):
                    vew = ebuf[sb, j, pl.ds(g * 16, 16)]
                    for r in range(16):
                        sc = vew[r]
                        row = g * 16 + r
                        for q in range(d // 16):
                            slots[slot, row, pl.ds(q * 16, 16)] = (
                                slots[slot, row, pl.ds(q * 16, 16)] * sc)
                    return c3

                lax.fori_loop(0, K // 16, scale, 0)
            pltpu.sync_copy(slots.at[slot], acc.at[cbuf.at[sb, j]],
                            add=True)

            @pl.when(j + NB < SUP)
            def _():
                pltpu.async_copy(table.at[rbuf.at[sb, j + NB]],
                                 slots.at[slot], gsem)

            return c2

        lax.fori_loop(0, SUP, chunk, 0)

        @pl.when(s + 2 < nsup)
        def _():
            idx_issue(s + 2, sb)

        return carry

    lax.fori_loop(0, nsup, do_super, 0)


def _writeback(acc, out_hbm, bounce, sid):
    off = sid * TROWS
    for b in range(WB):
        pltpu.sync_copy(acc.at[pl.ds(off + b * K, K)], bounce)
        pltpu.sync_copy(bounce, out_hbm.at[pl.ds(off + b * K, K)])


def _deg_body(sim_col, sim_ew, dist_col, dist_ew, com_col, com_ew,
              deg_s, deg_d, deg_c, dega, degb, cidx, vals, bounce, ssem):
    cid = lax.axis_index("c")
    sid = lax.axis_index("s")

    _zero_fill_1d(bounce, TROWS)
    pltpu.sync_copy(bounce, dega.at[pl.ds(sid * TROWS, TROWS)])
    pltpu.sync_copy(bounce, degb.at[pl.ds(sid * TROWS, TROWS)])
    plsc.subcore_barrier()

    DK = 8

    def scalar_pass(col2d, ew2d, acc, nchunks):
        pltpu.sync_copy(col2d.at[pl.ds(sid * nchunks, nchunks)],
                        cidx.at[pl.ds(0, nchunks)])
        pltpu.sync_copy(ew2d.at[pl.ds(sid * nchunks, nchunks)],
                        vals.at[pl.ds(0, nchunks)])

        def chunk(i, carry):
            pltpu.async_copy(vals.at[i], acc.at[cidx.at[i]], ssem,
                             add=True)

            @pl.when(i >= DK)
            def _():
                pltpu.make_async_copy(vals.at[0], acc.at[cidx.at[0]],
                                      ssem).wait()

            return carry

        lax.fori_loop(0, nchunks, chunk, 0)

        def drain(i, carry):
            pltpu.make_async_copy(vals.at[0], acc.at[cidx.at[0]],
                                  ssem).wait()
            return carry

        lax.fori_loop(0, DK, drain, 0)

    @pl.when(cid == 0)
    def _():
        scalar_pass(sim_col, sim_ew, dega, PT // K)

    @pl.when(cid == 1)
    def _():
        scalar_pass(dist_col, dist_ew, dega, PT // K)
        scalar_pass(com_col, com_ew, degb, PTC16 // K)

    plsc.subcore_barrier()
    off = sid * TROWS

    @pl.when(cid == 0)
    def _():
        pltpu.sync_copy(dega.at[pl.ds(off, TROWS)], bounce)
        pltpu.sync_copy(bounce, deg_s.at[pl.ds(off, TROWS)])

    @pl.when(cid == 1)
    def _():
        pltpu.sync_copy(dega.at[pl.ds(off, TROWS)], bounce)
        pltpu.sync_copy(bounce, deg_d.at[pl.ds(off, TROWS)])
        pltpu.sync_copy(degb.at[pl.ds(off, TROWS)], bounce)
        pltpu.sync_copy(bounce, deg_c.at[pl.ds(off, TROWS)])


def _sc_deg(sim_col, sim_ew, dist_col, dist_ew, com_col, com_ew):
    return pl.kernel(
        _deg_body,
        out_type=(jax.ShapeDtypeStruct((NP,), f32),
                  jax.ShapeDtypeStruct((NP,), f32),
                  jax.ShapeDtypeStruct((NP,), f32)),
        mesh=_mesh(),
        scratch_types=[
            pltpu.VMEM_SHARED((NP,), f32),
            pltpu.VMEM_SHARED((NP,), f32),
            pltpu.VMEM((PT // K, K), jnp.int32),
            pltpu.VMEM((PT // K, K), f32),
            pltpu.VMEM((TROWS,), f32),
            pltpu.SemaphoreType.DMA,
        ],
    )(sim_col, sim_ew, dist_col, dist_ew, com_col, com_ew)


def _l1_body(sim_row, sim_col, sim_ew, dist_row, dist_col, y1, y2,
             agg_sim, agg_dist, acc, rbuf, cbuf, ebuf, slots, gsem, isem):
    cid = lax.axis_index("c")
    sid = lax.axis_index("s")

    zb = slots.at[0]
    _zero_fill(zb, K, 128)
    _spmem_zero(acc, zb, sid)
    plsc.subcore_barrier()

    nchunks = PT // K

    @pl.when(cid == 0)
    def _():
        _edge_pass(sim_row, sim_col, sim_ew, y1, acc, rbuf, cbuf, ebuf,
                   slots, gsem, isem, sid, nchunks, 128)

    @pl.when(cid == 1)
    def _():
        _edge_pass(dist_row, dist_col, None, y2, acc, rbuf, cbuf, ebuf,
                   slots, gsem, isem, sid, nchunks, 128)

    plsc.subcore_barrier()

    @pl.when(cid == 0)
    def _():
        _writeback(acc, agg_sim, zb, sid)

    @pl.when(cid == 1)
    def _():
        _writeback(acc, agg_dist, zb, sid)


def _sc_l1(sim_row, sim_col, sim_ew, dist_row, dist_col, y1, y2):
    return pl.kernel(
        _l1_body,
        out_type=(jax.ShapeDtypeStruct((NP, 128), f32),
                  jax.ShapeDtypeStruct((NP, 128), f32)),
        mesh=_mesh(),
        scratch_types=[
            pltpu.VMEM_SHARED((NP, 128), f32),
            pltpu.VMEM((2, SUP, K), jnp.int32),
            pltpu.VMEM((2, SUP, K), jnp.int32),
            pltpu.VMEM((2, SUP, K), f32),
            pltpu.VMEM((NB, K, 128), f32),
            pltpu.SemaphoreType.DMA,
            pltpu.SemaphoreType.DMA,
        ],
    )(sim_row, sim_col, sim_ew, dist_row, dist_col, y1, y2)


def _l2_body(sim_row, sim_col, sim_ew, dist_row, dist_col, com_row,
             com_col, y3, y4, u, agg3, agg4, aggc0, aggc1,
             acc32, acc64, rbuf, cbuf, ebuf, slots32, slots64, gsem,
             isem):
    cid = lax.axis_index("c")
    sid = lax.axis_index("s")

    zb32 = slots32.at[0]
    zb64 = slots64.at[0]
    _zero_fill(zb32, K, 32)
    _zero_fill(zb64, K, 64)
    _spmem_zero(acc32, zb32, sid)
    _spmem_zero(acc64, zb64, sid)
    plsc.subcore_barrier()

    nchunks = PT // K

    @pl.when(cid == 0)
    def _():
        _edge_pass(sim_row, sim_col, sim_ew, y3, acc32, rbuf, cbuf,
                   ebuf, slots32, gsem, isem, sid, nchunks, 32)

    @pl.when(cid == 1)
    def _():
        _edge_pass(dist_row, dist_col, None, y4, acc32, rbuf, cbuf,
                   ebuf, slots32, gsem, isem, sid, nchunks, 32)

    ctile = cid * NT + sid
    _edge_pass(com_row, com_col, None, u, acc64, rbuf, cbuf, None,
               slots64, gsem, isem, ctile, PTC32 // K, 64)

    plsc.subcore_barrier()

    @pl.when(cid == 0)
    def _():
        _writeback(acc32, agg3, zb32, sid)
        _writeback(acc64, aggc0, zb64, sid)

    @pl.when(cid == 1)
    def _():
        _writeback(acc32, agg4, zb32, sid)
        _writeback(acc64, aggc1, zb64, sid)


def _sc_l2(sim_row, sim_col, sim_ew, dist_row, dist_col, com_row,
           com_col, y3, y4, u):
    return pl.kernel(
        _l2_body,
        out_type=(jax.ShapeDtypeStruct((NP, 32), f32),
                  jax.ShapeDtypeStruct((NP, 32), f32),
                  jax.ShapeDtypeStruct((NP, 64), f32),
                  jax.ShapeDtypeStruct((NP, 64), f32)),
        mesh=_mesh(),
        scratch_types=[
            pltpu.VMEM_SHARED((NP, 32), f32),
            pltpu.VMEM_SHARED((NP, 64), f32),
            pltpu.VMEM((2, SUP, K), jnp.int32),
            pltpu.VMEM((2, SUP, K), jnp.int32),
            pltpu.VMEM((2, SUP, K), f32),
            pltpu.VMEM((NB, K, 32), f32),
            pltpu.VMEM((NB, K, 64), f32),
            pltpu.SemaphoreType.DMA,
            pltpu.SemaphoreType.DMA,
        ],
        compiler_params=pltpu.CompilerParams(use_tc_tiling_on_sc=False),
    )(sim_row, sim_col, sim_ew, dist_row, dist_col, com_row, com_col,
      y3, y4, u)


_MB = NP // 8


def _tc1_body(deg_s, deg_d, deg_c, x, w1, w2, b1, b2, xadt, wpro, xatac,
              watac, bpa, dinv_s, dinv_d, dinv_c, y1, y2, sl1, sl2, u,
              slu):
    ds = lax.rsqrt(jnp.maximum(deg_s[...] + 1.0, 1e-12))
    dd = lax.rsqrt(jnp.maximum(deg_d[...] + 1.0, 1e-12))
    dc = lax.rsqrt(jnp.maximum(deg_c[...] + 1.0, 1e-12))
    dinv_s[...] = ds
    dinv_d[...] = dd
    dinv_c[...] = dc
    xw1 = jnp.dot(x[...], w1[...], preferred_element_type=f32)
    xw2 = jnp.dot(x[...], w2[...], preferred_element_type=f32)
    y1[...] = xw1 * ds
    y2[...] = xw2 * dd
    sl1[...] = xw1 * (ds * ds) + b1[...][None, :]
    sl2[...] = xw2 * (dd * dd) + b2[...][None, :]
    za = jnp.dot(xadt[...], wpro[...], preferred_element_type=f32)
    zt = jnp.dot(xatac[...], watac[...], preferred_element_type=f32)
    zc = jnp.concatenate([za, zt], axis=1)
    u[...] = zc * dc
    slu[...] = zc * (dc * dc) + bpa[...][None, :]


def _tc1(deg_s, deg_d, deg_c, x, w1, w2, b1, b2, xadt, wpro, xatac,
         watac, bpa):
    blk = lambda *s: pl.BlockSpec(s, lambda i: (0,) * len(s))
    row = lambda d: pl.BlockSpec((_MB, d), lambda i: (i, 0))
    return pl.pallas_call(
        _tc1_body,
        grid=(8,),
        in_specs=[
            row(1), row(1), row(1),
            row(128), blk(128, 128), blk(128, 128), blk(128,), blk(128,),
            row(16), blk(16, 32), row(64), blk(64, 32), blk(64,),
        ],
        out_specs=[
            row(1), row(1), row(1),
            row(128), row(128), row(128), row(128), row(64), row(64),
        ],
        out_shape=[
            jax.ShapeDtypeStruct((NP, 1), f32),
            jax.ShapeDtypeStruct((NP, 1), f32),
            jax.ShapeDtypeStruct((NP, 1), f32),
            jax.ShapeDtypeStruct((NP, 128), f32),
            jax.ShapeDtypeStruct((NP, 128), f32),
            jax.ShapeDtypeStruct((NP, 128), f32),
            jax.ShapeDtypeStruct((NP, 128), f32),
            jax.ShapeDtypeStruct((NP, 64), f32),
            jax.ShapeDtypeStruct((NP, 64), f32),
        ],
    )(deg_s, deg_d, deg_c, x, w1, w2, b1, b2, xadt, wpro, xatac, watac,
      bpa)


def _tc2_body(dinv_s, dinv_d, agg_s, agg_d, sl1, sl2, wsim, wdist, bsim,
              bdist, y3, y4, sl3, sl4):
    ds = dinv_s[...]
    dd = dinv_d[...]
    xs = jnp.maximum(agg_s[...] * ds + sl1[...], 0.0)
    xd = jnp.maximum(agg_d[...] * dd + sl2[...], 0.0)
    zs = jnp.dot(xs, wsim[...], preferred_element_type=f32)
    zd = jnp.dot(xd, wdist[...], preferred_element_type=f32)
    y3[...] = zs * ds
    y4[...] = zd * dd
    sl3[...] = zs * (ds * ds) + bsim[...][None, :]
    sl4[...] = zd * (dd * dd) + bdist[...][None, :]


def _tc2(dinv_s, dinv_d, agg_s, agg_d, sl1, sl2, wsim, wdist, bsim,
         bdist):
    blk = lambda *s: pl.BlockSpec(s, lambda i: (0,) * len(s))
    row = lambda d: pl.BlockSpec((_MB, d), lambda i: (i, 0))
    return pl.pallas_call(
        _tc2_body,
        grid=(8,),
        in_specs=[
            row(1), row(1),
            row(128), row(128), row(128), row(128),
            blk(128, 32), blk(128, 32), blk(32,), blk(32,),
        ],
        out_specs=[row(32), row(32), row(32), row(32)],
        out_shape=[jax.ShapeDtypeStruct((NP, 32), f32)] * 4,
    )(dinv_s, dinv_d, agg_s, agg_d, sl1, sl2, wsim, wdist, bsim, bdist)


def _tc3_body(dinv_s, dinv_d, dinv_c, agg3, agg4, aggc0, aggc1, sl3, sl4,
              slu, wf, bf, x_sim, x_dist, fused, pro, atac):
    ds = dinv_s[...]
    dd = dinv_d[...]
    dc = dinv_c[...]
    xsim = agg3[...] * ds + sl3[...]
    xdist = agg4[...] * dd + sl4[...]
    pa = (aggc0[...] + aggc1[...]) * dc + slu[...]
    x_sim[...] = xsim
    x_dist[...] = xdist
    pro[...] = pa[:, :32]
    atac[...] = pa[:, 32:]
    comb = jnp.concatenate([xsim, xdist, pa], axis=1)
    fused[...] = jnp.dot(comb, wf[...], preferred_element_type=f32) \
        + bf[...][None, :]


def _tc3(dinv_s, dinv_d, dinv_c, agg3, agg4, aggc0, aggc1, sl3, sl4, slu,
         wf, bf):
    blk = lambda *s: pl.BlockSpec(s, lambda i: (0,) * len(s))
    row = lambda d: pl.BlockSpec((_MB, d), lambda i: (i, 0))
    return pl.pallas_call(
        _tc3_body,
        grid=(8,),
        in_specs=[
            row(1), row(1), row(1),
            row(32), row(32), row(64), row(64), row(32), row(32), row(64),
            blk(128, 32), blk(32,),
        ],
        out_specs=[row(32), row(32), row(32), row(32), row(32)],
        out_shape=[jax.ShapeDtypeStruct((NP, 32), f32)] * 5,
    )(dinv_s, dinv_d, dinv_c, agg3, agg4, aggc0, aggc1, sl3, sl4, slu,
      wf, bf)


def kernel(x_RNA, x_ADT, x_ATAC, sim_edge_index, sim_edge_weight,
           dist_edge_index, dist_edge_weight, common_edge_index,
           common_edge_weight, W_rna1, b_rna1, W_rna2, b_rna2, W_pro,
           b_pro, W_atac, b_atac, W_sim, b_sim, W_dist, b_dist, W_fuse,
           b_fuse):
    pad = NP - N
    xp = jnp.pad(x_RNA, ((0, pad), (0, 0)))
    xadt = jnp.pad(x_ADT, ((0, pad), (0, 0)))
    xatac = jnp.pad(x_ATAC, ((0, pad), (0, 0)))
    def _tiles(a, ntiles, per, fill):
        orig = a.shape[0] // ntiles
        a2 = jnp.pad(a.reshape(ntiles, orig), ((0, 0), (0, per - orig)),
                     constant_values=fill)
        return a2.reshape(ntiles * (per // K), K)

    sim_row = _tiles(sim_edge_index[0], NT, PT, 0)
    sim_col = _tiles(sim_edge_index[1], NT, PT, NP - 1)
    sim_ew = _tiles(sim_edge_weight, NT, PT, 0.0)
    dist_row = _tiles(dist_edge_index[0], NT, PT, 0)
    dist_col = _tiles(dist_edge_index[1], NT, PT, NP - 1)
    dist_ew = _tiles(dist_edge_weight, NT, PT, 0.0)
    com_row = _tiles(common_edge_index[0], 2 * NT, PTC32, 0)
    com_col = _tiles(common_edge_index[1], 2 * NT, PTC32, NP - 1)
    com_col16 = _tiles(common_edge_index[1], NT, PTC16, NP - 1)
    com_ew16 = _tiles(common_edge_weight, NT, PTC16, 0.0)
    bpa = jnp.concatenate([b_pro, b_atac])

    DBG_DEG = False
    if DBG_DEG:
        deg_s = jnp.zeros((NP,), f32).at[sim_col.reshape(-1)].add(
            sim_ew.reshape(-1))
        deg_d = jnp.zeros((NP,), f32).at[dist_col.reshape(-1)].add(
            dist_ew.reshape(-1))
        deg_c = jnp.zeros((NP,), f32).at[com_col16.reshape(-1)].add(
            com_ew16.reshape(-1))
    else:
        deg_s, deg_d, deg_c = _sc_deg(sim_col, sim_ew, dist_col, dist_ew,
                                      com_col16, com_ew16)

    dinv_s, dinv_d, dinv_c, y1, y2, sl1, sl2, u, slu = _tc1(
        deg_s[:, None], deg_d[:, None], deg_c[:, None], xp, W_rna1,
        W_rna2, b_rna1, b_rna2, xadt, W_pro, xatac, W_atac, bpa)

    agg_sim, agg_dist = _sc_l1(sim_row, sim_col, sim_ew, dist_row,
                               dist_col, y1, y2)

    y3, y4, sl3, sl4 = _tc2(dinv_s, dinv_d, agg_sim, agg_dist, sl1, sl2,
                            W_sim, W_dist, b_sim, b_dist)

    DBG_L2 = False
    if DBG_L2:
        agg3 = jnp.zeros((NP, 32), f32).at[sim_col.reshape(-1)].add(
            y3[sim_row.reshape(-1)] * sim_ew.reshape(-1)[:, None])
        agg4 = jnp.zeros((NP, 32), f32).at[dist_col.reshape(-1)].add(
            y4[dist_row.reshape(-1)])
        aggc0 = jnp.zeros((NP, 64), f32).at[com_col.reshape(-1)].add(
            u[com_row.reshape(-1)])
        aggc1 = jnp.zeros((NP, 64), f32)
    else:
        agg3, agg4, aggc0, aggc1 = _sc_l2(sim_row, sim_col, sim_ew,
                                          dist_row, dist_col, com_row,
                                          com_col, y3, y4, u)

    x_sim, x_dist, fused, pro, atac = _tc3(dinv_s, dinv_d, dinv_c, agg3,
                                           agg4, aggc0, aggc1, sl3, sl4,
                                           slu, W_fuse, b_fuse)

    return (x_sim[:N], x_dist[:N], fused[:N], pro[:N], atac[:N])


# speedup vs baseline: 1.2064x; 1.0304x over previous
import functools

import jax
import jax.numpy as jnp
from jax import lax
from jax.experimental import pallas as pl
from jax.experimental.pallas import tpu as pltpu
from jax.experimental.pallas import tpu_sc as plsc

N = 10000
NP = 10240
E = 320000
EC = 64000
K = 128
NT = 16
TROWS = NP // NT
WB = TROWS // K
PT = 20480
PTC16 = 4096
PTC32 = 2048

f32 = jnp.float32


def _mesh():
    return plsc.VectorSubcoreMesh(core_axis_name="c", subcore_axis_name="s")


def _zero_fill(buf, rows, cols):
    z = jnp.zeros((16,), f32)

    def body(r, carry):
        for j in range(cols // 16):
            buf[r, pl.ds(j * 16, 16)] = z
        return carry

    lax.fori_loop(0, rows, body, 0)


def _zero_fill_1d(buf, n):
    z = jnp.zeros((16,), f32)

    def body(r, carry):
        buf[pl.ds(r * 16, 16)] = z
        return carry

    lax.fori_loop(0, n // 16, body, 0)


def _spmem_zero(acc, zbuf, sid):
    off = sid * TROWS
    for b in range(WB):
        pltpu.sync_copy(zbuf, acc.at[pl.ds(off + b * K, K)])


NB = 2
SUP = 16
PIPE = False


def _edge_pass(row2d, col2d, ew2d, table, acc, rbuf, cbuf, ebuf, slots,
               gsem, isem, tile, nchunks, d):
    cb = tile * nchunks
    nsup = nchunks // SUP

    if not PIPE:

        def do_super_sync(s, carry):
            sb = lax.rem(s, 2)
            base = cb + s * SUP
            pltpu.sync_copy(row2d.at[pl.ds(base, SUP)], rbuf.at[sb])
            pltpu.sync_copy(col2d.at[pl.ds(base, SUP)], cbuf.at[sb])
            if ew2d is not None:
                pltpu.sync_copy(ew2d.at[pl.ds(base, SUP)], ebuf.at[sb])

            def chunk(j, c2):
                pltpu.sync_copy(table.at[rbuf.at[sb, j]], slots.at[0])
                if ew2d is not None:

                    def scale(g, c3):
                        vew = ebuf[sb, j, pl.ds(g * 16, 16)]
                        for r in range(16):
                            sc = vew[r]
                            row = g * 16 + r
                            for q in range(d // 16):
                                slots[0, row, pl.ds(q * 16, 16)] = (
                                    slots[0, row, pl.ds(q * 16, 16)] * sc)
                        return c3

                    lax.fori_loop(0, K // 16, scale, 0)
                pltpu.sync_copy(slots.at[0], acc.at[cbuf.at[sb, j]],
                                add=True)
                return c2

            lax.fori_loop(0, SUP, chunk, 0)
            return carry

        lax.fori_loop(0, nsup, do_super_sync, 0)
        return

    def idx_issue(s, sb):
        base = cb + s * SUP
        pltpu.async_copy(row2d.at[pl.ds(base, SUP)], rbuf.at[sb], isem)
        pltpu.async_copy(col2d.at[pl.ds(base, SUP)], cbuf.at[sb], isem)
        if ew2d is not None:
            pltpu.async_copy(ew2d.at[pl.ds(base, SUP)], ebuf.at[sb], isem)

    def idx_wait(sb):
        pltpu.make_async_copy(row2d.at[pl.ds(cb, SUP)], rbuf.at[sb],
                              isem).wait()
        pltpu.make_async_copy(col2d.at[pl.ds(cb, SUP)], cbuf.at[sb],
                              isem).wait()
        if ew2d is not None:
            pltpu.make_async_copy(ew2d.at[pl.ds(cb, SUP)], ebuf.at[sb],
                                  isem).wait()

    for s in range(min(2, nsup)):
        idx_issue(s, s)

    def do_super(s, carry):
        sb = lax.rem(s, 2)
        idx_wait(sb)
        for b in range(NB):
            pltpu.async_copy(table.at[rbuf.at[sb, b]], slots.at[b], gsem)

        def chunk(j, c2):
            slot = lax.rem(j, NB)
            pltpu.make_async_copy(table.at[rbuf.at[sb, j]],
                                  slots.at[slot], gsem).wait()
            if ew2d is not None:

                def scale(g, c3):
                    vew = ebuf[sb, j, pl.ds(g * 16, 16)]
                    for r in range(16):
                        sc = vew[r]
                        row = g * 16 + r
                        for q in range(d // 16):
                            slots[slot, row, pl.ds(q * 16, 16)] = (
                                slots[slot, row, pl.ds(q * 16, 16)] * sc)
                    return c3

                lax.fori_loop(0, K // 16, scale, 0)
            pltpu.sync_copy(slots.at[slot], acc.at[cbuf.at[sb, j]],
                            add=True)

            @pl.when(j + NB < SUP)
            def _():
                pltpu.async_copy(table.at[rbuf.at[sb, j + NB]],
                                 slots.at[slot], gsem)

            return c2

        lax.fori_loop(0, SUP, chunk, 0)

        @pl.when(s + 2 < nsup)
        def _():
            idx_issue(s + 2, sb)

        return carry

    lax.fori_loop(0, nsup, do_super, 0)


G = 4
SUP8 = 2 * G


def _edge_pass_grouped(row2d, col2d, ew2d, table, acc, rbuf, cbuf, ebuf,
                       slots, gsem, ssem, isem, tile, nchunks, d):
    cb = tile * nchunks
    nsup = nchunks // SUP8

    def idx_issue(sup, sb):
        base = cb + sup * SUP8
        pltpu.async_copy(row2d.at[pl.ds(base, SUP8)], rbuf.at[sb], isem)
        pltpu.async_copy(col2d.at[pl.ds(base, SUP8)], cbuf.at[sb], isem)
        if ew2d is not None:
            pltpu.async_copy(ew2d.at[pl.ds(base, SUP8)], ebuf.at[sb],
                             isem)

    def idx_wait(sb):
        pltpu.make_async_copy(row2d.at[pl.ds(cb, SUP8)], rbuf.at[sb],
                              isem).wait()
        pltpu.make_async_copy(col2d.at[pl.ds(cb, SUP8)], cbuf.at[sb],
                              isem).wait()
        if ew2d is not None:
            pltpu.make_async_copy(ew2d.at[pl.ds(cb, SUP8)], ebuf.at[sb],
                                  isem).wait()

    def fire_gathers(sb, row0, set0):
        for b in range(G):
            pltpu.async_copy(table.at[rbuf.at[sb, row0 + b]],
                             slots.at[set0 * G + b], gsem)

    def wait_gathers(sb, row0, set0):
        for b in range(G):
            pltpu.make_async_copy(table.at[rbuf.at[sb, row0 + b]],
                                  slots.at[set0 * G + b], gsem).wait()

    def fire_scatters(sb, row0, set0):
        for b in range(G):
            pltpu.async_copy(slots.at[set0 * G + b],
                             acc.at[cbuf.at[sb, row0 + b]], ssem,
                             add=True)

    def drain_scatters(set0):
        for b in range(G):
            pltpu.make_async_copy(slots.at[set0 * G + b],
                                  acc.at[cbuf.at[0, 0]], ssem).wait()

    def scale_group(sb, row0, set0):
        if ew2d is None:
            return

        def scale(g2, c3):
            for b in range(G):
                vew = ebuf[sb, row0 + b, pl.ds(g2 * 16, 16)]
                slot = set0 * G + b
                for r in range(16):
                    sc = vew[r]
                    row = g2 * 16 + r
                    for q in range(d // 16):
                        slots[slot, row, pl.ds(q * 16, 16)] = (
                            slots[slot, row, pl.ds(q * 16, 16)] * sc)
            return c3

        lax.fori_loop(0, K // 16, scale, 0)

    idx_issue(0, 0)
    idx_wait(0)
    fire_gathers(0, 0, 0)

    def body(sb_i, carry):
        sbuf = lax.rem(sb_i, 2)
        wait_gathers(sbuf, 0, 0)
        scale_group(sbuf, 0, 0)

        @pl.when(sb_i > 0)
        def _():
            drain_scatters(1)

        @pl.when(sb_i + 1 < nsup)
        def _():
            idx_issue(sb_i + 1, 1 - sbuf)

        fire_gathers(sbuf, G, 1)
        fire_scatters(sbuf, 0, 0)
        wait_gathers(sbuf, G, 1)
        scale_group(sbuf, G, 1)
        drain_scatters(0)

        @pl.when(sb_i + 1 < nsup)
        def _():
            idx_wait(1 - sbuf)
            fire_gathers(1 - sbuf, 0, 0)

        fire_scatters(sbuf, G, 1)
        return carry

    lax.fori_loop(0, nsup, body, 0)
    drain_scatters(1)


def _writeback(acc, out_hbm, bounce, sid):
    off = sid * TROWS
    for b in range(WB):
        pltpu.sync_copy(acc.at[pl.ds(off + b * K, K)], bounce)
        pltpu.sync_copy(bounce, out_hbm.at[pl.ds(off + b * K, K)])


def _deg_body(sim_col, sim_ew, dist_col, dist_ew, com_col, com_ew,
              deg_s, deg_d, deg_c, dega, degb, cidx, vals, bounce, ssem):
    cid = lax.axis_index("c")
    sid = lax.axis_index("s")

    _zero_fill_1d(bounce, TROWS)
    pltpu.sync_copy(bounce, dega.at[pl.ds(sid * TROWS, TROWS)])
    pltpu.sync_copy(bounce, degb.at[pl.ds(sid * TROWS, TROWS)])
    plsc.subcore_barrier()

    DK = 8

    def scalar_pass(col2d, ew2d, acc, nchunks):
        pltpu.sync_copy(col2d.at[pl.ds(sid * nchunks, nchunks)],
                        cidx.at[pl.ds(0, nchunks)])
        pltpu.sync_copy(ew2d.at[pl.ds(sid * nchunks, nchunks)],
                        vals.at[pl.ds(0, nchunks)])

        def chunk(i, carry):
            pltpu.async_copy(vals.at[i], acc.at[cidx.at[i]], ssem,
                             add=True)

            @pl.when(i >= DK)
            def _():
                pltpu.make_async_copy(vals.at[0], acc.at[cidx.at[0]],
                                      ssem).wait()

            return carry

        lax.fori_loop(0, nchunks, chunk, 0)

        def drain(i, carry):
            pltpu.make_async_copy(vals.at[0], acc.at[cidx.at[0]],
                                  ssem).wait()
            return carry

        lax.fori_loop(0, DK, drain, 0)

    @pl.when(cid == 0)
    def _():
        scalar_pass(sim_col, sim_ew, dega, PT // K)

    @pl.when(cid == 1)
    def _():
        scalar_pass(dist_col, dist_ew, dega, PT // K)
        scalar_pass(com_col, com_ew, degb, PTC16 // K)

    plsc.subcore_barrier()
    off = sid * TROWS

    @pl.when(cid == 0)
    def _():
        pltpu.sync_copy(dega.at[pl.ds(off, TROWS)], bounce)
        pltpu.sync_copy(bounce, deg_s.at[pl.ds(off, TROWS)])

    @pl.when(cid == 1)
    def _():
        pltpu.sync_copy(dega.at[pl.ds(off, TROWS)], bounce)
        pltpu.sync_copy(bounce, deg_d.at[pl.ds(off, TROWS)])
        pltpu.sync_copy(degb.at[pl.ds(off, TROWS)], bounce)
        pltpu.sync_copy(bounce, deg_c.at[pl.ds(off, TROWS)])


def _sc_deg(sim_col, sim_ew, dist_col, dist_ew, com_col, com_ew):
    return pl.kernel(
        _deg_body,
        out_type=(jax.ShapeDtypeStruct((NP,), f32),
                  jax.ShapeDtypeStruct((NP,), f32),
                  jax.ShapeDtypeStruct((NP,), f32)),
        mesh=_mesh(),
        scratch_types=[
            pltpu.VMEM_SHARED((NP,), f32),
            pltpu.VMEM_SHARED((NP,), f32),
            pltpu.VMEM((PT // K, K), jnp.int32),
            pltpu.VMEM((PT // K, K), f32),
            pltpu.VMEM((TROWS,), f32),
            pltpu.SemaphoreType.DMA,
        ],
    )(sim_col, sim_ew, dist_col, dist_ew, com_col, com_ew)


def _l1_body(sim_row, sim_col, sim_ew, dist_row, dist_col,
             y1a, y1b, y2a, y2b,
             agg_sa, agg_sb, agg_da, agg_db,
             acc, rbuf, cbuf, ebuf, slots, gsem, ssem, isem):
    cid = lax.axis_index("c")
    sid = lax.axis_index("s")
    nchunks = PT // K
    zb = slots.at[0]

    for half in range(2):
        ta = (y1a, y1b)[half]
        tb = (y2a, y2b)[half]
        oa = (agg_sa, agg_sb)[half]
        ob = (agg_da, agg_db)[half]
        _zero_fill(zb, K, 64)
        _spmem_zero(acc, zb, sid)
        plsc.subcore_barrier()

        @pl.when(cid == 0)
        def _():
            _edge_pass_grouped(sim_row, sim_col, sim_ew, ta, acc, rbuf,
                               cbuf, ebuf, slots, gsem, ssem, isem, sid,
                               nchunks, 64)

        @pl.when(cid == 1)
        def _():
            _edge_pass_grouped(dist_row, dist_col, None, tb, acc, rbuf,
                               cbuf, ebuf, slots, gsem, ssem, isem, sid,
                               nchunks, 64)

        plsc.subcore_barrier()

        @pl.when(cid == 0)
        def _():
            _writeback(acc, oa, zb, sid)

        @pl.when(cid == 1)
        def _():
            _writeback(acc, ob, zb, sid)

        plsc.subcore_barrier()


def _sc_l1(sim_row, sim_col, sim_ew, dist_row, dist_col, y1a, y1b, y2a,
           y2b):
    return pl.kernel(
        _l1_body,
        out_type=tuple(jax.ShapeDtypeStruct((NP, 64), f32)
                       for _ in range(4)),
        mesh=_mesh(),
        scratch_types=[
            pltpu.VMEM_SHARED((NP, 64), f32),
            pltpu.VMEM((2, SUP8, K), jnp.int32),
            pltpu.VMEM((2, SUP8, K), jnp.int32),
            pltpu.VMEM((2, SUP8, K), f32),
            pltpu.VMEM((2 * G, K, 64), f32),
            pltpu.SemaphoreType.DMA,
            pltpu.SemaphoreType.DMA,
            pltpu.SemaphoreType.DMA,
        ],
        compiler_params=pltpu.CompilerParams(use_tc_tiling_on_sc=False),
    )(sim_row, sim_col, sim_ew, dist_row, dist_col, y1a, y1b, y2a, y2b)


def _l2_body(sim_row, sim_col, sim_ew, dist_row, dist_col, com_row,
             com_col, y3, y4, u, agg3, agg4, aggc0, aggc1,
             acc32, acc64, rbuf, cbuf, ebuf, slots32, slots64, gsem,
             isem):
    cid = lax.axis_index("c")
    sid = lax.axis_index("s")

    zb32 = slots32.at[0]
    zb64 = slots64.at[0]
    _zero_fill(zb32, K, 32)
    _zero_fill(zb64, K, 64)
    _spmem_zero(acc32, zb32, sid)
    _spmem_zero(acc64, zb64, sid)
    plsc.subcore_barrier()

    nchunks = PT // K

    @pl.when(cid == 0)
    def _():
        _edge_pass(sim_row, sim_col, sim_ew, y3, acc32, rbuf, cbuf,
                   ebuf, slots32, gsem, isem, sid, nchunks, 32)

    @pl.when(cid == 1)
    def _():
        _edge_pass(dist_row, dist_col, None, y4, acc32, rbuf, cbuf,
                   ebuf, slots32, gsem, isem, sid, nchunks, 32)

    ctile = cid * NT + sid
    _edge_pass(com_row, com_col, None, u, acc64, rbuf, cbuf, None,
               slots64, gsem, isem, ctile, PTC32 // K, 64)

    plsc.subcore_barrier()

    @pl.when(cid == 0)
    def _():
        _writeback(acc32, agg3, zb32, sid)
        _writeback(acc64, aggc0, zb64, sid)

    @pl.when(cid == 1)
    def _():
        _writeback(acc32, agg4, zb32, sid)
        _writeback(acc64, aggc1, zb64, sid)


def _sc_l2(sim_row, sim_col, sim_ew, dist_row, dist_col, com_row,
           com_col, y3, y4, u):
    return pl.kernel(
        _l2_body,
        out_type=(jax.ShapeDtypeStruct((NP, 32), f32),
                  jax.ShapeDtypeStruct((NP, 32), f32),
                  jax.ShapeDtypeStruct((NP, 64), f32),
                  jax.ShapeDtypeStruct((NP, 64), f32)),
        mesh=_mesh(),
        scratch_types=[
            pltpu.VMEM_SHARED((NP, 32), f32),
            pltpu.VMEM_SHARED((NP, 64), f32),
            pltpu.VMEM((2, SUP, K), jnp.int32),
            pltpu.VMEM((2, SUP, K), jnp.int32),
            pltpu.VMEM((2, SUP, K), f32),
            pltpu.VMEM((NB, K, 32), f32),
            pltpu.VMEM((NB, K, 64), f32),
            pltpu.SemaphoreType.DMA,
            pltpu.SemaphoreType.DMA,
        ],
        compiler_params=pltpu.CompilerParams(use_tc_tiling_on_sc=False),
    )(sim_row, sim_col, sim_ew, dist_row, dist_col, com_row, com_col,
      y3, y4, u)


_MB = NP // 8


def _tc1_body(deg_s, deg_d, deg_c, x, w1, w2, b1, b2, xadt, wpro, xatac,
              watac, bpa, dinv_s, dinv_d, dinv_c, y1a, y1b, y2a, y2b,
              sl1, sl2, u, slu):
    ds = lax.rsqrt(jnp.maximum(deg_s[...] + 1.0, 1e-12))
    dd = lax.rsqrt(jnp.maximum(deg_d[...] + 1.0, 1e-12))
    dc = lax.rsqrt(jnp.maximum(deg_c[...] + 1.0, 1e-12))
    dinv_s[...] = ds
    dinv_d[...] = dd
    dinv_c[...] = dc
    xw1 = jnp.dot(x[...], w1[...], preferred_element_type=f32)
    xw2 = jnp.dot(x[...], w2[...], preferred_element_type=f32)
    t1 = xw1 * ds
    t2 = xw2 * dd
    y1a[...] = t1[:, :64]
    y1b[...] = t1[:, 64:]
    y2a[...] = t2[:, :64]
    y2b[...] = t2[:, 64:]
    sl1[...] = xw1 * (ds * ds) + b1[...][None, :]
    sl2[...] = xw2 * (dd * dd) + b2[...][None, :]
    za = jnp.dot(xadt[...], wpro[...], preferred_element_type=f32)
    zt = jnp.dot(xatac[...], watac[...], preferred_element_type=f32)
    zc = jnp.concatenate([za, zt], axis=1)
    u[...] = zc * dc
    slu[...] = zc * (dc * dc) + bpa[...][None, :]


def _tc1(deg_s, deg_d, deg_c, x, w1, w2, b1, b2, xadt, wpro, xatac,
         watac, bpa):
    blk = lambda *s: pl.BlockSpec(s, lambda i: (0,) * len(s))
    row = lambda d: pl.BlockSpec((_MB, d), lambda i: (i, 0))
    return pl.pallas_call(
        _tc1_body,
        grid=(8,),
        in_specs=[
            row(1), row(1), row(1),
            row(128), blk(128, 128), blk(128, 128), blk(128,), blk(128,),
            row(16), blk(16, 32), row(64), blk(64, 32), blk(64,),
        ],
        out_specs=[
            row(1), row(1), row(1),
            row(64), row(64), row(64), row(64),
            row(128), row(128), row(64), row(64),
        ],
        out_shape=[
            jax.ShapeDtypeStruct((NP, 1), f32),
            jax.ShapeDtypeStruct((NP, 1), f32),
            jax.ShapeDtypeStruct((NP, 1), f32),
            jax.ShapeDtypeStruct((NP, 64), f32),
            jax.ShapeDtypeStruct((NP, 64), f32),
            jax.ShapeDtypeStruct((NP, 64), f32),
            jax.ShapeDtypeStruct((NP, 64), f32),
            jax.ShapeDtypeStruct((NP, 128), f32),
            jax.ShapeDtypeStruct((NP, 128), f32),
            jax.ShapeDtypeStruct((NP, 64), f32),
            jax.ShapeDtypeStruct((NP, 64), f32),
        ],
    )(deg_s, deg_d, deg_c, x, w1, w2, b1, b2, xadt, wpro, xatac, watac,
      bpa)


def _tc2_body(dinv_s, dinv_d, agg_sa, agg_sb, agg_da, agg_db, sl1, sl2,
              wsim, wdist, bsim, bdist, y3, y4, sl3, sl4):
    ds = dinv_s[...]
    dd = dinv_d[...]
    agg_s = jnp.concatenate([agg_sa[...], agg_sb[...]], axis=1)
    agg_d = jnp.concatenate([agg_da[...], agg_db[...]], axis=1)
    xs = jnp.maximum(agg_s * ds + sl1[...], 0.0)
    xd = jnp.maximum(agg_d * dd + sl2[...], 0.0)
    zs = jnp.dot(xs, wsim[...], preferred_element_type=f32)
    zd = jnp.dot(xd, wdist[...], preferred_element_type=f32)
    y3[...] = zs * ds
    y4[...] = zd * dd
    sl3[...] = zs * (ds * ds) + bsim[...][None, :]
    sl4[...] = zd * (dd * dd) + bdist[...][None, :]


def _tc2(dinv_s, dinv_d, agg_sa, agg_sb, agg_da, agg_db, sl1, sl2, wsim,
         wdist, bsim, bdist):
    blk = lambda *s: pl.BlockSpec(s, lambda i: (0,) * len(s))
    row = lambda d: pl.BlockSpec((_MB, d), lambda i: (i, 0))
    return pl.pallas_call(
        _tc2_body,
        grid=(8,),
        in_specs=[
            row(1), row(1),
            row(64), row(64), row(64), row(64), row(128), row(128),
            blk(128, 32), blk(128, 32), blk(32,), blk(32,),
        ],
        out_specs=[row(32), row(32), row(32), row(32)],
        out_shape=[jax.ShapeDtypeStruct((NP, 32), f32)] * 4,
    )(dinv_s, dinv_d, agg_sa, agg_sb, agg_da, agg_db, sl1, sl2, wsim,
      wdist, bsim, bdist)


def _tc3_body(dinv_s, dinv_d, dinv_c, agg3, agg4, aggc0, aggc1, sl3, sl4,
              slu, wf, bf, x_sim, x_dist, fused, pro, atac):
    ds = dinv_s[...]
    dd = dinv_d[...]
    dc = dinv_c[...]
    xsim = agg3[...] * ds + sl3[...]
    xdist = agg4[...] * dd + sl4[...]
    pa = (aggc0[...] + aggc1[...]) * dc + slu[...]
    x_sim[...] = xsim
    x_dist[...] = xdist
    pro[...] = pa[:, :32]
    atac[...] = pa[:, 32:]
    comb = jnp.concatenate([xsim, xdist, pa], axis=1)
    fused[...] = jnp.dot(comb, wf[...], preferred_element_type=f32) \
        + bf[...][None, :]


def _tc3(dinv_s, dinv_d, dinv_c, agg3, agg4, aggc0, aggc1, sl3, sl4, slu,
         wf, bf):
    blk = lambda *s: pl.BlockSpec(s, lambda i: (0,) * len(s))
    row = lambda d: pl.BlockSpec((_MB, d), lambda i: (i, 0))
    return pl.pallas_call(
        _tc3_body,
        grid=(8,),
        in_specs=[
            row(1), row(1), row(1),
            row(32), row(32), row(64), row(64), row(32), row(32), row(64),
            blk(128, 32), blk(32,),
        ],
        out_specs=[row(32), row(32), row(32), row(32), row(32)],
        out_shape=[jax.ShapeDtypeStruct((NP, 32), f32)] * 5,
    )(dinv_s, dinv_d, dinv_c, agg3, agg4, aggc0, aggc1, sl3, sl4, slu,
      wf, bf)


def kernel(x_RNA, x_ADT, x_ATAC, sim_edge_index, sim_edge_weight,
           dist_edge_index, dist_edge_weight, common_edge_index,
           common_edge_weight, W_rna1, b_rna1, W_rna2, b_rna2, W_pro,
           b_pro, W_atac, b_atac, W_sim, b_sim, W_dist, b_dist, W_fuse,
           b_fuse):
    pad = NP - N
    xp = jnp.pad(x_RNA, ((0, pad), (0, 0)))
    xadt = jnp.pad(x_ADT, ((0, pad), (0, 0)))
    xatac = jnp.pad(x_ATAC, ((0, pad), (0, 0)))
    def _tiles(a, ntiles, per, fill):
        orig = a.shape[0] // ntiles
        a2 = jnp.pad(a.reshape(ntiles, orig), ((0, 0), (0, per - orig)),
                     constant_values=fill)
        return a2.reshape(ntiles * (per // K), K)

    sim_row = _tiles(sim_edge_index[0], NT, PT, 0)
    sim_col = _tiles(sim_edge_index[1], NT, PT, NP - 1)
    sim_ew = _tiles(sim_edge_weight, NT, PT, 0.0)
    dist_row = _tiles(dist_edge_index[0], NT, PT, 0)
    dist_col = _tiles(dist_edge_index[1], NT, PT, NP - 1)
    dist_ew = _tiles(dist_edge_weight, NT, PT, 0.0)
    com_row = _tiles(common_edge_index[0], 2 * NT, PTC32, 0)
    com_col = _tiles(common_edge_index[1], 2 * NT, PTC32, NP - 1)
    com_col16 = _tiles(common_edge_index[1], NT, PTC16, NP - 1)
    com_ew16 = _tiles(common_edge_weight, NT, PTC16, 0.0)
    bpa = jnp.concatenate([b_pro, b_atac])

    DBG_DEG = False
    if DBG_DEG:
        deg_s = jnp.zeros((NP,), f32).at[sim_col.reshape(-1)].add(
            sim_ew.reshape(-1))
        deg_d = jnp.zeros((NP,), f32).at[dist_col.reshape(-1)].add(
            dist_ew.reshape(-1))
        deg_c = jnp.zeros((NP,), f32).at[com_col16.reshape(-1)].add(
            com_ew16.reshape(-1))
    else:
        deg_s, deg_d, deg_c = _sc_deg(sim_col, sim_ew, dist_col, dist_ew,
                                      com_col16, com_ew16)

    (dinv_s, dinv_d, dinv_c, y1a, y1b, y2a, y2b, sl1, sl2, u,
     slu) = _tc1(
        deg_s[:, None], deg_d[:, None], deg_c[:, None], xp, W_rna1,
        W_rna2, b_rna1, b_rna2, xadt, W_pro, xatac, W_atac, bpa)

    agg_sa, agg_sb, agg_da, agg_db = _sc_l1(sim_row, sim_col, sim_ew,
                                            dist_row, dist_col, y1a,
                                            y1b, y2a, y2b)

    y3, y4, sl3, sl4 = _tc2(dinv_s, dinv_d, agg_sa, agg_sb, agg_da,
                            agg_db, sl1, sl2, W_sim, W_dist, b_sim,
                            b_dist)

    DBG_L2 = False
    if DBG_L2:
        agg3 = jnp.zeros((NP, 32), f32).at[sim_col.reshape(-1)].add(
            y3[sim_row.reshape(-1)] * sim_ew.reshape(-1)[:, None])
        agg4 = jnp.zeros((NP, 32), f32).at[dist_col.reshape(-1)].add(
            y4[dist_row.reshape(-1)])
        aggc0 = jnp.zeros((NP, 64), f32).at[com_col.reshape(-1)].add(
            u[com_row.reshape(-1)])
        aggc1 = jnp.zeros((NP, 64), f32)
    else:
        agg3, agg4, aggc0, aggc1 = _sc_l2(sim_row, sim_col, sim_ew,
                                          dist_row, dist_col, com_row,
                                          com_col, y3, y4, u)

    x_sim, x_dist, fused, pro, atac = _tc3(dinv_s, dinv_d, dinv_c, agg3,
                                           agg4, aggc0, aggc1, sl3, sl4,
                                           slu, W_fuse, b_fuse)

    return (x_sim[:N], x_dist[:N], fused[:N], pro[:N], atac[:N])
